# baseline (speedup 1.0000x reference)
import functools

import jax
import jax.numpy as jnp
from jax import lax
from jax.experimental import pallas as pl
from jax.experimental.pallas import tpu as pltpu
from jax.experimental.pallas import tpu_sc as plsc

NUM_USERS = 1000000
NUM_ITEMS = 1000000
LATENT_DIM = 32
BATCH = 16384
BLK = 128
RPB = BLK // LATENT_DIM

_INFO = plsc.get_sparse_core_info()
NC, NS, L = _INFO.num_cores, _INFO.num_subcores, _INFO.num_lanes
NW = NC * NS
BPW = BATCH // NW
CH = 128
NCH = BPW // CH
GPC = CH // L

_MESH = plsc.VectorSubcoreMesh(core_axis_name="c", subcore_axis_name="s")


@functools.partial(
    pl.kernel,
    mesh=_MESH,
    compiler_params=pltpu.CompilerParams(needs_layout_passes=False),
    out_type=jax.ShapeDtypeStruct((BATCH,), jnp.float32),
    scratch_types=[
        pltpu.VMEM((BPW,), jnp.int32),
        pltpu.VMEM((BPW,), jnp.int32),
        pltpu.VMEM((NCH, CH), jnp.int32),
        pltpu.VMEM((NCH, CH), jnp.int32),
        pltpu.VMEM((CH, BLK), jnp.float32),
        pltpu.VMEM((CH, BLK), jnp.float32),
        pltpu.VMEM((CH, BLK), jnp.float32),
        pltpu.VMEM((CH, BLK), jnp.float32),
        pltpu.VMEM((BPW,), jnp.float32),
        pltpu.SemaphoreType.DMA,
        pltpu.SemaphoreType.DMA,
        pltpu.SemaphoreType.DMA,
        pltpu.SemaphoreType.DMA,
    ],
)
def _var_mf_sc(users_h, items_h, ut_h, it_h, out_h,
               uidx_v, iidx_v, uq_v, iq_v, ub0, ub1, ib0, ib1, gout_v,
               sem_u0, sem_u1, sem_i0, sem_i1):
    wid = lax.axis_index("s") * NC + lax.axis_index("c")
    base = wid * BPW

    pltpu.sync_copy(users_h.at[pl.ds(base, BPW)], uidx_v)
    pltpu.sync_copy(items_h.at[pl.ds(base, BPW)], iidx_v)

    def qloop(t, _):
        c = t // (CH // L)
        o = (t % (CH // L)) * L
        uq_v[c, pl.ds(o, L)] = lax.shift_right_logical(
            uidx_v[pl.ds(t * L, L)], 2)
        iq_v[c, pl.ds(o, L)] = lax.shift_right_logical(
            iidx_v[pl.ds(t * L, L)], 2)
        return 0

    lax.fori_loop(0, BPW // L, qloop, 0)

    lane = lax.iota(jnp.int32, L)
    ubufs = (ub0, ub1)
    ibufs = (ib0, ib1)
    usems = (sem_u0, sem_u1)
    isems = (sem_i0, sem_i1)

    def fire(c):
        cu = pltpu.async_copy(ut_h.at[uq_v.at[c]], ubufs[c % 2], usems[c % 2])
        ci = pltpu.async_copy(it_h.at[iq_v.at[c]], ibufs[c % 2], isems[c % 2])
        return cu, ci

    inflight = fire(0)
    for c in range(NCH):
        nxt = fire(c + 1) if c + 1 < NCH else None
        inflight[0].wait()
        inflight[1].wait()
        ub = ubufs[c % 2]
        ib = ibufs[c % 2]

        def group(g, _):
            goff = c * CH + g * L
            ui = uidx_v[pl.ds(goff, L)]
            ii = iidx_v[pl.ds(goff, L)]
            ucol = lax.shift_left(ui & (RPB - 1), 5)
            icol = lax.shift_left(ii & (RPB - 1), 5)
            rows = g * L + lane
            num = jnp.zeros((L,), jnp.float32)
            den = jnp.zeros((L,), jnp.float32)
            for j in range(LATENT_DIM):
                cu = plsc.load_gather(ub, [rows, ucol + j])
                ci = plsc.load_gather(ib, [rows, icol + j])
                e = jnp.exp(ci)
                den = den + e
                num = num + e / (1.0 + jnp.exp(-cu))
            gout_v[pl.ds(goff, L)] = num / den
            return 0

        lax.fori_loop(0, GPC, group, 0)
        inflight = nxt

    pltpu.sync_copy(gout_v, out_h.at[pl.ds(base, BPW)])


def kernel(users, items, user_table, item_table):
    users = users.astype(jnp.int32)
    items = items.astype(jnp.int32)
    ut = user_table.reshape(NUM_USERS * LATENT_DIM // BLK, BLK)
    it = item_table.reshape(NUM_ITEMS * LATENT_DIM // BLK, BLK)
    return _var_mf_sc(users, items, ut, it)


# speedup vs baseline: 1.0027x; 1.0027x over previous
import functools

import jax
import jax.numpy as jnp
from jax import lax
from jax.experimental import pallas as pl
from jax.experimental.pallas import tpu as pltpu
from jax.experimental.pallas import tpu_sc as plsc

NUM_USERS = 1000000
NUM_ITEMS = 1000000
LATENT_DIM = 32
BATCH = 16384
BLK = 128
RPB = BLK // LATENT_DIM

_INFO = plsc.get_sparse_core_info()
NC, NS, L = _INFO.num_cores, _INFO.num_subcores, _INFO.num_lanes
NW = NC * NS
BPW = BATCH // NW
CH = 128
NCH = BPW // CH
GPC = CH // L

_MESH = plsc.VectorSubcoreMesh(core_axis_name="c", subcore_axis_name="s")


@functools.partial(
    pl.kernel,
    mesh=_MESH,
    compiler_params=pltpu.CompilerParams(needs_layout_passes=False,
                                         use_tc_tiling_on_sc=True),
    out_type=jax.ShapeDtypeStruct((BATCH,), jnp.float32),
    scratch_types=[
        pltpu.VMEM((BPW,), jnp.int32),
        pltpu.VMEM((BPW,), jnp.int32),
        pltpu.VMEM((NCH, CH), jnp.int32),
        pltpu.VMEM((NCH, CH), jnp.int32),
        pltpu.VMEM((CH, BLK), jnp.float32),
        pltpu.VMEM((CH, BLK), jnp.float32),
        pltpu.VMEM((CH, BLK), jnp.float32),
        pltpu.VMEM((CH, BLK), jnp.float32),
        pltpu.VMEM((BPW,), jnp.float32),
        pltpu.SemaphoreType.DMA,
        pltpu.SemaphoreType.DMA,
        pltpu.SemaphoreType.DMA,
        pltpu.SemaphoreType.DMA,
    ],
)
def _var_mf_sc(users_h, items_h, ut_h, it_h, out_h,
               uidx_v, iidx_v, uq_v, iq_v, ub0, ub1, ib0, ib1, gout_v,
               sem_u0, sem_u1, sem_i0, sem_i1):
    wid = lax.axis_index("s") * NC + lax.axis_index("c")
    base = wid * BPW

    pltpu.sync_copy(users_h.at[pl.ds(base, BPW)], uidx_v)
    pltpu.sync_copy(items_h.at[pl.ds(base, BPW)], iidx_v)

    def qloop(t, _):
        c = t // (CH // L)
        o = (t % (CH // L)) * L
        uq_v[c, pl.ds(o, L)] = lax.shift_right_logical(
            uidx_v[pl.ds(t * L, L)], 2)
        iq_v[c, pl.ds(o, L)] = lax.shift_right_logical(
            iidx_v[pl.ds(t * L, L)], 2)
        return 0

    lax.fori_loop(0, BPW // L, qloop, 0)

    lane = lax.iota(jnp.int32, L)
    ubufs = (ub0, ub1)
    ibufs = (ib0, ib1)
    usems = (sem_u0, sem_u1)
    isems = (sem_i0, sem_i1)

    def fire(c):
        cu = pltpu.async_copy(ut_h.at[uq_v.at[c]], ubufs[c % 2], usems[c % 2])
        ci = pltpu.async_copy(it_h.at[iq_v.at[c]], ibufs[c % 2], isems[c % 2])
        return cu, ci

    inflight = fire(0)
    for c in range(NCH):
        nxt = fire(c + 1) if c + 1 < NCH else None
        inflight[0].wait()
        inflight[1].wait()
        ub = ubufs[c % 2]
        ib = ibufs[c % 2]

        def group(g, _):
            goff = c * CH + g * L
            ui = uidx_v[pl.ds(goff, L)]
            ii = iidx_v[pl.ds(goff, L)]
            ucol = lax.shift_left(ui & (RPB - 1), 5)
            icol = lax.shift_left(ii & (RPB - 1), 5)
            rows = g * L + lane
            num = jnp.zeros((L,), jnp.float32)
            den = jnp.zeros((L,), jnp.float32)
            for j in range(LATENT_DIM):
                cu = plsc.load_gather(ub, [rows, ucol + j])
                ci = plsc.load_gather(ib, [rows, icol + j])
                e = jnp.exp(ci)
                den = den + e
                num = num + e / (1.0 + jnp.exp(-cu))
            gout_v[pl.ds(goff, L)] = num / den
            return 0

        lax.fori_loop(0, GPC, group, 0)
        inflight = nxt

    pltpu.sync_copy(gout_v, out_h.at[pl.ds(base, BPW)])


def kernel(users, items, user_table, item_table):
    users = users.astype(jnp.int32)
    items = items.astype(jnp.int32)
    ut = user_table.reshape(NUM_USERS * LATENT_DIM // BLK, BLK)
    it = item_table.reshape(NUM_ITEMS * LATENT_DIM // BLK, BLK)
    return _var_mf_sc(users, items, ut, it)


# speedup vs baseline: 2.8786x; 2.8709x over previous
import functools

import jax
import jax.numpy as jnp
from jax import lax
from jax.experimental import pallas as pl
from jax.experimental.pallas import tpu as pltpu
from jax.experimental.pallas import tpu_sc as plsc

NUM_USERS = 1000000
NUM_ITEMS = 1000000
LATENT_DIM = 32
BATCH = 16384
WIN = 128

_INFO = plsc.get_sparse_core_info()
NC, NS, L = _INFO.num_cores, _INFO.num_subcores, _INFO.num_lanes
NW = NC * NS
BPW = BATCH // NW
GRP = BPW // L

_MESH = plsc.VectorSubcoreMesh(core_axis_name="c", subcore_axis_name="s")


@functools.partial(
    pl.kernel,
    mesh=_MESH,
    compiler_params=pltpu.CompilerParams(needs_layout_passes=False,
                                         use_tc_tiling_on_sc=True),
    out_type=jax.ShapeDtypeStruct((BATCH,), jnp.float32),
    scratch_types=[
        pltpu.VMEM((BPW,), jnp.int32),
        pltpu.VMEM((BPW,), jnp.int32),
        pltpu.VMEM((LATENT_DIM, WIN), jnp.float32),
        pltpu.VMEM((LATENT_DIM, WIN), jnp.float32),
        pltpu.VMEM((LATENT_DIM, WIN), jnp.float32),
        pltpu.VMEM((LATENT_DIM, WIN), jnp.float32),
        pltpu.VMEM((L * L,), jnp.float32),
        pltpu.VMEM((L * L,), jnp.float32),
        pltpu.VMEM((BPW,), jnp.float32),
        pltpu.SemaphoreType.DMA,
        pltpu.SemaphoreType.DMA,
        pltpu.SemaphoreType.DMA,
        pltpu.SemaphoreType.DMA,
    ],
)
def _var_mf_sc(users_h, items_h, utt_h, itt_h, out_h,
               uidx_v, iidx_v, ua_v, ub_v, ia_v, ib_v,
               nbuf_v, dbuf_v, gout_v,
               sem_ua, sem_ub, sem_ia, sem_ib):
    wid = lax.axis_index("s") * NC + lax.axis_index("c")
    base = wid * BPW

    pltpu.sync_copy(users_h.at[pl.ds(base, BPW)], uidx_v)
    pltpu.sync_copy(items_h.at[pl.ds(base, BPW)], iidx_v)

    lane = lax.iota(jnp.int32, L)

    def sget(vref, r):
        chunk = vref[pl.ds((r // L) * L, L)]
        return jnp.sum(jnp.where(lane == r % L, chunk, 0))

    ubufs = (ua_v, ub_v)
    ibufs = (ia_v, ib_v)
    usems = (sem_ua, sem_ub)
    isems = (sem_ia, sem_ib)

    def fire(r, slot):
        rr = min(r, BPW - 1) if isinstance(r, int) else lax.min(r, BPW - 1)
        tu = lax.shift_right_logical(sget(uidx_v, rr), 7)
        ti = lax.shift_right_logical(sget(iidx_v, rr), 7)
        pltpu.async_copy(
            utt_h.at[:, pl.ds(pl.multiple_of(tu * WIN, WIN), WIN)],
            ubufs[slot], usems[slot])
        pltpu.async_copy(
            itt_h.at[:, pl.ds(pl.multiple_of(ti * WIN, WIN), WIN)],
            ibufs[slot], isems[slot])

    def wait(slot):
        pltpu.make_async_copy(utt_h.at[:, pl.ds(0, WIN)],
                              ubufs[slot], usems[slot]).wait()
        pltpu.make_async_copy(itt_h.at[:, pl.ds(0, WIN)],
                              ibufs[slot], isems[slot]).wait()

    fire(0, 0)
    fire(1, 1)

    rowbase = lane * L

    def group(g, _):
        for rr in range(L):
            slot = rr % 2
            r = g * L + rr
            wait(slot)
            uchunk = uidx_v[pl.ds(g * L, L)]
            ichunk = iidx_v[pl.ds(g * L, L)]
            sel = jnp.full((L,), rr, jnp.int32)
            luv = jnp.take(uchunk, sel) & (WIN - 1)
            liv = jnp.take(ichunk, sel) & (WIN - 1)
            uv0 = plsc.load_gather(ubufs[slot], [lane, luv])
            uv1 = plsc.load_gather(ubufs[slot], [lane + L, luv])
            iv0 = plsc.load_gather(ibufs[slot], [lane, liv])
            iv1 = plsc.load_gather(ibufs[slot], [lane + L, liv])
            fire(r + 2, slot)
            e0 = jnp.exp(iv0)
            e1 = jnp.exp(iv1)
            s0 = 1.0 / (1.0 + jnp.exp(-uv0))
            s1 = 1.0 / (1.0 + jnp.exp(-uv1))
            nbuf_v[pl.ds(rr * L, L)] = e0 * s0 + e1 * s1
            dbuf_v[pl.ds(rr * L, L)] = e0 + e1
        num = jnp.zeros((L,), jnp.float32)
        den = jnp.zeros((L,), jnp.float32)
        for k in range(L):
            num = num + plsc.load_gather(nbuf_v, [rowbase + k])
            den = den + plsc.load_gather(dbuf_v, [rowbase + k])
        gout_v[pl.ds(g * L, L)] = num / den
        return 0

    lax.fori_loop(0, GRP, group, 0)

    wait(0)
    wait(1)

    pltpu.sync_copy(gout_v, out_h.at[pl.ds(base, BPW)])


def kernel(users, items, user_table, item_table):
    users = users.astype(jnp.int32)
    items = items.astype(jnp.int32)
    return _var_mf_sc(users, items, user_table.T, item_table.T)


# speedup vs baseline: 3.8922x; 1.3521x over previous
import functools

import jax
import jax.numpy as jnp
from jax import lax
from jax.experimental import pallas as pl
from jax.experimental.pallas import tpu as pltpu
from jax.experimental.pallas import tpu_sc as plsc

NUM_USERS = 1000000
NUM_ITEMS = 1000000
LATENT_DIM = 32
BATCH = 16384
WIN = 128

_INFO = plsc.get_sparse_core_info()
NC, NS, L = _INFO.num_cores, _INFO.num_subcores, _INFO.num_lanes
NW = NC * NS
BPW = BATCH // NW
GRP = BPW // L

_MESH = plsc.VectorSubcoreMesh(core_axis_name="c", subcore_axis_name="s")


@functools.partial(
    pl.kernel,
    mesh=_MESH,
    compiler_params=pltpu.CompilerParams(needs_layout_passes=False,
                                         use_tc_tiling_on_sc=True),
    out_type=jax.ShapeDtypeStruct((BATCH,), jnp.float32),
    scratch_types=[
        pltpu.VMEM((BPW,), jnp.int32),
        pltpu.VMEM((BPW,), jnp.int32),
        pltpu.VMEM((LATENT_DIM, WIN), jnp.float32),
        pltpu.VMEM((LATENT_DIM, WIN), jnp.float32),
        pltpu.VMEM((LATENT_DIM, WIN), jnp.float32),
        pltpu.VMEM((LATENT_DIM, WIN), jnp.float32),
        pltpu.VMEM((LATENT_DIM, WIN), jnp.float32),
        pltpu.VMEM((LATENT_DIM, WIN), jnp.float32),
        pltpu.VMEM((LATENT_DIM, WIN), jnp.float32),
        pltpu.VMEM((LATENT_DIM, WIN), jnp.float32),
        pltpu.VMEM((L * L,), jnp.float32),
        pltpu.VMEM((L * L,), jnp.float32),
        pltpu.VMEM((BPW,), jnp.float32),
        pltpu.SemaphoreType.DMA,
        pltpu.SemaphoreType.DMA,
        pltpu.SemaphoreType.DMA,
        pltpu.SemaphoreType.DMA,
        pltpu.SemaphoreType.DMA,
        pltpu.SemaphoreType.DMA,
        pltpu.SemaphoreType.DMA,
        pltpu.SemaphoreType.DMA,
    ],
)
def _var_mf_sc(users_h, items_h, utt_h, itt_h, out_h,
               uidx_v, iidx_v, u0_v, u1_v, u2_v, u3_v, i0_v, i1_v, i2_v, i3_v,
               nbuf_v, dbuf_v, gout_v,
               sem_u0, sem_u1, sem_u2, sem_u3,
               sem_i0, sem_i1, sem_i2, sem_i3):
    wid = lax.axis_index("s") * NC + lax.axis_index("c")
    base = wid * BPW

    pltpu.sync_copy(users_h.at[pl.ds(base, BPW)], uidx_v)
    pltpu.sync_copy(items_h.at[pl.ds(base, BPW)], iidx_v)

    lane = lax.iota(jnp.int32, L)

    def sget(vref, r):
        chunk = vref[pl.ds((r // L) * L, L)]
        return jnp.sum(jnp.where(lane == r % L, chunk, 0))

    NSLOT = 4
    ubufs = (u0_v, u1_v, u2_v, u3_v)
    ibufs = (i0_v, i1_v, i2_v, i3_v)
    usems = (sem_u0, sem_u1, sem_u2, sem_u3)
    isems = (sem_i0, sem_i1, sem_i2, sem_i3)

    def fire(r, slot):
        rr = min(r, BPW - 1) if isinstance(r, int) else lax.min(r, BPW - 1)
        tu = lax.shift_right_logical(sget(uidx_v, rr), 7)
        ti = lax.shift_right_logical(sget(iidx_v, rr), 7)
        pltpu.async_copy(
            utt_h.at[:, pl.ds(pl.multiple_of(tu * WIN, WIN), WIN)],
            ubufs[slot], usems[slot])
        pltpu.async_copy(
            itt_h.at[:, pl.ds(pl.multiple_of(ti * WIN, WIN), WIN)],
            ibufs[slot], isems[slot])

    def wait(slot):
        pltpu.make_async_copy(utt_h.at[:, pl.ds(0, WIN)],
                              ubufs[slot], usems[slot]).wait()
        pltpu.make_async_copy(itt_h.at[:, pl.ds(0, WIN)],
                              ibufs[slot], isems[slot]).wait()

    for p in range(NSLOT):
        fire(p, p)

    rowbase = lane * L

    def group(g, _):
        for rr in range(L):
            slot = rr % NSLOT
            r = g * L + rr
            wait(slot)
            uchunk = uidx_v[pl.ds(g * L, L)]
            ichunk = iidx_v[pl.ds(g * L, L)]
            sel = jnp.full((L,), rr, jnp.int32)
            luv = jnp.take(uchunk, sel) & (WIN - 1)
            liv = jnp.take(ichunk, sel) & (WIN - 1)
            uv0 = plsc.load_gather(ubufs[slot], [lane, luv])
            uv1 = plsc.load_gather(ubufs[slot], [lane + L, luv])
            iv0 = plsc.load_gather(ibufs[slot], [lane, liv])
            iv1 = plsc.load_gather(ibufs[slot], [lane + L, liv])
            fire(r + NSLOT, slot)
            e0 = jnp.exp(iv0)
            e1 = jnp.exp(iv1)
            s0 = 1.0 / (1.0 + jnp.exp(-uv0))
            s1 = 1.0 / (1.0 + jnp.exp(-uv1))
            nbuf_v[pl.ds(rr * L, L)] = e0 * s0 + e1 * s1
            dbuf_v[pl.ds(rr * L, L)] = e0 + e1
        num = jnp.zeros((L,), jnp.float32)
        den = jnp.zeros((L,), jnp.float32)
        for k in range(L):
            num = num + plsc.load_gather(nbuf_v, [rowbase + k])
            den = den + plsc.load_gather(dbuf_v, [rowbase + k])
        gout_v[pl.ds(g * L, L)] = num / den
        return 0

    lax.fori_loop(0, GRP, group, 0)

    for p in range(NSLOT):
        wait(p)

    pltpu.sync_copy(gout_v, out_h.at[pl.ds(base, BPW)])


def kernel(users, items, user_table, item_table):
    users = users.astype(jnp.int32)
    items = items.astype(jnp.int32)
    return _var_mf_sc(users, items, user_table.T, item_table.T)


# speedup vs baseline: 3.9503x; 1.0149x over previous
import functools

import jax
import jax.numpy as jnp
from jax import lax
from jax.experimental import pallas as pl
from jax.experimental.pallas import tpu as pltpu
from jax.experimental.pallas import tpu_sc as plsc

NUM_USERS = 1000000
NUM_ITEMS = 1000000
LATENT_DIM = 32
BATCH = 16384
WIN = 128

_INFO = plsc.get_sparse_core_info()
NC, NS, L = _INFO.num_cores, _INFO.num_subcores, _INFO.num_lanes
NW = NC * NS
BPW = BATCH // NW
GRP = BPW // L

_MESH = plsc.VectorSubcoreMesh(core_axis_name="c", subcore_axis_name="s")


@functools.partial(
    pl.kernel,
    mesh=_MESH,
    compiler_params=pltpu.CompilerParams(needs_layout_passes=False,
                                         use_tc_tiling_on_sc=True),
    out_type=jax.ShapeDtypeStruct((BATCH,), jnp.float32),
    scratch_types=[
        pltpu.VMEM((BPW,), jnp.int32),
        pltpu.VMEM((BPW,), jnp.int32),
        pltpu.VMEM((LATENT_DIM, WIN), jnp.float32),
        pltpu.VMEM((LATENT_DIM, WIN), jnp.float32),
        pltpu.VMEM((LATENT_DIM, WIN), jnp.float32),
        pltpu.VMEM((LATENT_DIM, WIN), jnp.float32),
        pltpu.VMEM((LATENT_DIM, WIN), jnp.float32),
        pltpu.VMEM((LATENT_DIM, WIN), jnp.float32),
        pltpu.VMEM((LATENT_DIM, WIN), jnp.float32),
        pltpu.VMEM((LATENT_DIM, WIN), jnp.float32),
        pltpu.VMEM((LATENT_DIM, WIN), jnp.float32),
        pltpu.VMEM((LATENT_DIM, WIN), jnp.float32),
        pltpu.VMEM((LATENT_DIM, WIN), jnp.float32),
        pltpu.VMEM((LATENT_DIM, WIN), jnp.float32),
        pltpu.VMEM((LATENT_DIM, WIN), jnp.float32),
        pltpu.VMEM((LATENT_DIM, WIN), jnp.float32),
        pltpu.VMEM((LATENT_DIM, WIN), jnp.float32),
        pltpu.VMEM((LATENT_DIM, WIN), jnp.float32),
        pltpu.VMEM((L * L,), jnp.float32),
        pltpu.VMEM((L * L,), jnp.float32),
        pltpu.VMEM((BPW,), jnp.float32),
        pltpu.SemaphoreType.DMA,
        pltpu.SemaphoreType.DMA,
        pltpu.SemaphoreType.DMA,
        pltpu.SemaphoreType.DMA,
        pltpu.SemaphoreType.DMA,
        pltpu.SemaphoreType.DMA,
        pltpu.SemaphoreType.DMA,
        pltpu.SemaphoreType.DMA,
        pltpu.SemaphoreType.DMA,
        pltpu.SemaphoreType.DMA,
        pltpu.SemaphoreType.DMA,
        pltpu.SemaphoreType.DMA,
        pltpu.SemaphoreType.DMA,
        pltpu.SemaphoreType.DMA,
        pltpu.SemaphoreType.DMA,
        pltpu.SemaphoreType.DMA,
    ],
)
def _var_mf_sc(users_h, items_h, utt_h, itt_h, out_h,
               uidx_v, iidx_v, u0_v, u1_v, u2_v, u3_v, u4_v, u5_v, u6_v, u7_v, i0_v, i1_v, i2_v, i3_v, i4_v, i5_v, i6_v, i7_v,
               nbuf_v, dbuf_v, gout_v,
               sem_u0, sem_u1, sem_u2, sem_u3, sem_u4, sem_u5, sem_u6, sem_u7,
               sem_i0, sem_i1, sem_i2, sem_i3, sem_i4, sem_i5, sem_i6, sem_i7):
    wid = lax.axis_index("s") * NC + lax.axis_index("c")
    base = wid * BPW

    pltpu.sync_copy(users_h.at[pl.ds(base, BPW)], uidx_v)
    pltpu.sync_copy(items_h.at[pl.ds(base, BPW)], iidx_v)

    lane = lax.iota(jnp.int32, L)

    def sget(vref, r):
        chunk = vref[pl.ds((r // L) * L, L)]
        return jnp.sum(jnp.where(lane == r % L, chunk, 0))

    NSLOT = 8
    ubufs = (u0_v, u1_v, u2_v, u3_v, u4_v, u5_v, u6_v, u7_v)
    ibufs = (i0_v, i1_v, i2_v, i3_v, i4_v, i5_v, i6_v, i7_v)
    usems = (sem_u0, sem_u1, sem_u2, sem_u3, sem_u4, sem_u5, sem_u6, sem_u7)
    isems = (sem_i0, sem_i1, sem_i2, sem_i3, sem_i4, sem_i5, sem_i6, sem_i7)

    def fire(r, slot):
        rr = min(r, BPW - 1) if isinstance(r, int) else lax.min(r, BPW - 1)
        tu = lax.shift_right_logical(sget(uidx_v, rr), 7)
        ti = lax.shift_right_logical(sget(iidx_v, rr), 7)
        pltpu.async_copy(
            utt_h.at[:, pl.ds(pl.multiple_of(tu * WIN, WIN), WIN)],
            ubufs[slot], usems[slot])
        pltpu.async_copy(
            itt_h.at[:, pl.ds(pl.multiple_of(ti * WIN, WIN), WIN)],
            ibufs[slot], isems[slot])

    def wait(slot):
        pltpu.make_async_copy(utt_h.at[:, pl.ds(0, WIN)],
                              ubufs[slot], usems[slot]).wait()
        pltpu.make_async_copy(itt_h.at[:, pl.ds(0, WIN)],
                              ibufs[slot], isems[slot]).wait()

    for p in range(NSLOT):
        fire(p, p)

    rowbase = lane * L

    def group(g, _):
        for rr in range(L):
            slot = rr % NSLOT
            r = g * L + rr
            wait(slot)
            uchunk = uidx_v[pl.ds(g * L, L)]
            ichunk = iidx_v[pl.ds(g * L, L)]
            sel = jnp.full((L,), rr, jnp.int32)
            luv = jnp.take(uchunk, sel) & (WIN - 1)
            liv = jnp.take(ichunk, sel) & (WIN - 1)
            uv0 = plsc.load_gather(ubufs[slot], [lane, luv])
            uv1 = plsc.load_gather(ubufs[slot], [lane + L, luv])
            iv0 = plsc.load_gather(ibufs[slot], [lane, liv])
            iv1 = plsc.load_gather(ibufs[slot], [lane + L, liv])
            fire(r + NSLOT, slot)
            e0 = jnp.exp(iv0)
            e1 = jnp.exp(iv1)
            s0 = 1.0 / (1.0 + jnp.exp(-uv0))
            s1 = 1.0 / (1.0 + jnp.exp(-uv1))
            nbuf_v[pl.ds(rr * L, L)] = e0 * s0 + e1 * s1
            dbuf_v[pl.ds(rr * L, L)] = e0 + e1
        num = jnp.zeros((L,), jnp.float32)
        den = jnp.zeros((L,), jnp.float32)
        for k in range(L):
            num = num + plsc.load_gather(nbuf_v, [rowbase + k])
            den = den + plsc.load_gather(dbuf_v, [rowbase + k])
        gout_v[pl.ds(g * L, L)] = num / den
        return 0

    lax.fori_loop(0, GRP, group, 0)

    for p in range(NSLOT):
        wait(p)

    pltpu.sync_copy(gout_v, out_h.at[pl.ds(base, BPW)])


def kernel(users, items, user_table, item_table):
    users = users.astype(jnp.int32)
    items = items.astype(jnp.int32)
    return _var_mf_sc(users, items, user_table.T, item_table.T)
